# Initial kernel scaffold; baseline (speedup 1.0000x reference)
import jax
import jax.numpy as jnp
from jax.experimental import pallas as pl


def kernel(x, edge_index, edge_weight, W1, b1, W2, b2):
    raise NotImplementedError("write your pallas kernel here")


# speedup vs baseline: 39.4214x; 39.4214x over previous
import functools

import jax
import jax.numpy as jnp
from jax import lax
from jax.experimental import pallas as pl
from jax.experimental.pallas import tpu as pltpu
from jax.experimental.pallas import tpu_sc as plsc

N = 10000
E = 320000
D_IN = 128
D_HID = 16
D_OUT = 128

N_PAD = 10240
E_PAD = 327680
EROWS = E_PAD // 128
ROWS_PER_TILE = EROWS // 32
CHUNK_ROWS = 8
NCHUNK = ROWS_PER_TILE // CHUNK_ROWS
NODES_PER_TILE = N_PAD // 16

_MESH = plsc.VectorSubcoreMesh(core_axis_name="c", subcore_axis_name="s")


@functools.partial(
    pl.kernel,
    out_type=jax.ShapeDtypeStruct((2, N_PAD), jnp.float32),
    mesh=_MESH,
    scratch_types=[
        pltpu.VMEM_SHARED((N_PAD,), jnp.float32),
        pltpu.VMEM((CHUNK_ROWS, 128), jnp.int32),
        pltpu.VMEM((CHUNK_ROWS, 128), jnp.float32),
        pltpu.SemaphoreType.DMA,
    ],
    compiler_params=pltpu.CompilerParams(use_tc_tiling_on_sc=False),
)
def _sc_deg(dst_hbm, ew_hbm, zeros_hbm, out_hbm, acc_sh, dstv, eww, sem):
    c = lax.axis_index("c")
    s = lax.axis_index("s")
    wid = c * 16 + s
    node0 = s * NODES_PER_TILE

    pltpu.sync_copy(zeros_hbm, acc_sh.at[pl.ds(node0, NODES_PER_TILE)])
    plsc.subcore_barrier()

    row0 = wid * ROWS_PER_TILE

    def chunk(ci, _):
        base = row0 + ci * CHUNK_ROWS
        pltpu.sync_copy(dst_hbm.at[pl.ds(base, CHUNK_ROWS)], dstv)
        pltpu.sync_copy(ew_hbm.at[pl.ds(base, CHUNK_ROWS)], eww)
        descs = []
        for j in range(CHUNK_ROWS):
            descs.append(
                pltpu.async_copy(eww.at[j], acc_sh.at[dstv.at[j]], sem, add=True)
            )
        for d in descs:
            d.wait()
        return ()

    lax.fori_loop(0, NCHUNK, chunk, (), unroll=False)
    plsc.subcore_barrier()
    pltpu.sync_copy(
        acc_sh.at[pl.ds(node0, NODES_PER_TILE)],
        out_hbm.at[c, pl.ds(node0, NODES_PER_TILE)],
    )


@functools.partial(
    pl.kernel,
    out_type=jax.ShapeDtypeStruct((2, N_PAD, D_HID), jnp.float32),
    mesh=_MESH,
    scratch_types=[
        pltpu.VMEM_SHARED((N_PAD, D_HID), jnp.float32),
        pltpu.VMEM((CHUNK_ROWS, 128), jnp.int32),
        pltpu.VMEM((CHUNK_ROWS, 128), jnp.int32),
        pltpu.VMEM((CHUNK_ROWS, 128), jnp.float32),
        pltpu.VMEM((CHUNK_ROWS, 128, D_HID), jnp.float32),
        pltpu.SemaphoreType.DMA,
        pltpu.SemaphoreType.DMA,
    ],
    compiler_params=pltpu.CompilerParams(use_tc_tiling_on_sc=False),
)
def _sc_edge(src_hbm, dst_hbm, ew2_hbm, feat_hbm, zeros_hbm, out_hbm,
             acc_sh, srcv, dstv, eww, rows, gsem, ssem):
    c = lax.axis_index("c")
    s = lax.axis_index("s")
    wid = c * 16 + s
    node0 = s * NODES_PER_TILE

    pltpu.sync_copy(zeros_hbm, acc_sh.at[pl.ds(node0, NODES_PER_TILE)])
    plsc.subcore_barrier()

    row0 = wid * ROWS_PER_TILE

    def chunk(ci, _):
        base = row0 + ci * CHUNK_ROWS
        pltpu.sync_copy(src_hbm.at[pl.ds(base, CHUNK_ROWS)], srcv)
        pltpu.sync_copy(dst_hbm.at[pl.ds(base, CHUNK_ROWS)], dstv)
        pltpu.sync_copy(ew2_hbm.at[pl.ds(base, CHUNK_ROWS)], eww)
        gds = []
        for j in range(CHUNK_ROWS):
            gds.append(pltpu.async_copy(feat_hbm.at[srcv.at[j]], rows.at[j], gsem))
        for d in gds:
            d.wait()

        for j in range(CHUNK_ROWS):
            def scale16(g, _):
                e0 = g * 16
                wv = eww[j, pl.ds(e0, 16)]
                for k in range(16):
                    rows[j, e0 + k, :] = rows[j, e0 + k, :] * wv[k]
                return ()
            lax.fori_loop(0, 8, scale16, (), unroll=False)

        sds = []
        for j in range(CHUNK_ROWS):
            sds.append(
                pltpu.async_copy(rows.at[j], acc_sh.at[dstv.at[j]], ssem, add=True)
            )
        for d in sds:
            d.wait()
        return ()

    lax.fori_loop(0, NCHUNK, chunk, (), unroll=False)
    plsc.subcore_barrier()
    pltpu.sync_copy(
        acc_sh.at[pl.ds(node0, NODES_PER_TILE)],
        out_hbm.at[c, pl.ds(node0, NODES_PER_TILE)],
    )


_BLK = 1024
_GRID = N_PAD // _BLK


def _tc1_body(x_ref, w_ref, deg_ref, o_ref):
    dis = lax.rsqrt(deg_ref[0] + deg_ref[1] + 1.0)
    h = jnp.dot(x_ref[...], w_ref[...], preferred_element_type=jnp.float32)
    o_ref[...] = h * dis


def _tc2_body(p_ref, h_ref, deg_ref, b_ref, o_ref):
    dis = lax.rsqrt(deg_ref[0] + deg_ref[1] + 1.0)
    agg = dis * (p_ref[0] + p_ref[1] + h_ref[...]) + b_ref[...]
    o_ref[...] = dis * jnp.maximum(agg, 0.0)


def _tc3_body(q_ref, r_ref, deg_ref, w_ref, b_ref, o_ref):
    dis = lax.rsqrt(deg_ref[0] + deg_ref[1] + 1.0)
    agg = dis * (q_ref[0] + q_ref[1] + r_ref[...])
    y = jnp.dot(agg, w_ref[...], preferred_element_type=jnp.float32) + b_ref[...]
    m = jnp.max(y, axis=1, keepdims=True)
    lse = m + jnp.log(jnp.sum(jnp.exp(y - m), axis=1, keepdims=True))
    o_ref[...] = y - lse


def _tc1(x_pad, W1, deg_parts):
    return pl.pallas_call(
        _tc1_body,
        grid=(_GRID,),
        in_specs=[
            pl.BlockSpec((_BLK, D_IN), lambda i: (i, 0)),
            pl.BlockSpec((D_IN, D_HID), lambda i: (0, 0)),
            pl.BlockSpec((2, _BLK, 1), lambda i: (0, i, 0)),
        ],
        out_specs=pl.BlockSpec((_BLK, D_HID), lambda i: (i, 0)),
        out_shape=jax.ShapeDtypeStruct((N_PAD, D_HID), jnp.float32),
    )(x_pad, W1, deg_parts)


def _tc2(parts, h1p, deg_parts, b1):
    return pl.pallas_call(
        _tc2_body,
        grid=(_GRID,),
        in_specs=[
            pl.BlockSpec((2, _BLK, D_HID), lambda i: (0, i, 0)),
            pl.BlockSpec((_BLK, D_HID), lambda i: (i, 0)),
            pl.BlockSpec((2, _BLK, 1), lambda i: (0, i, 0)),
            pl.BlockSpec((1, D_HID), lambda i: (0, 0)),
        ],
        out_specs=pl.BlockSpec((_BLK, D_HID), lambda i: (i, 0)),
        out_shape=jax.ShapeDtypeStruct((N_PAD, D_HID), jnp.float32),
    )(parts, h1p, deg_parts, b1)


def _tc3(parts, rp, deg_parts, W2, b2):
    return pl.pallas_call(
        _tc3_body,
        grid=(_GRID,),
        in_specs=[
            pl.BlockSpec((2, _BLK, D_HID), lambda i: (0, i, 0)),
            pl.BlockSpec((_BLK, D_HID), lambda i: (i, 0)),
            pl.BlockSpec((2, _BLK, 1), lambda i: (0, i, 0)),
            pl.BlockSpec((D_HID, D_OUT), lambda i: (0, 0)),
            pl.BlockSpec((1, D_OUT), lambda i: (0, 0)),
        ],
        out_specs=pl.BlockSpec((_BLK, D_OUT), lambda i: (i, 0)),
        out_shape=jax.ShapeDtypeStruct((N_PAD, D_OUT), jnp.float32),
    )(parts, rp, deg_parts, W2, b2)


def kernel(x, edge_index, edge_weight, W1, b1, W2, b2):
    src = edge_index[0]
    dst = edge_index[1]

    npad = E_PAD - E
    pad_idx = N + (jnp.arange(npad, dtype=jnp.int32) % (N_PAD - N))
    src2 = jnp.concatenate([src, pad_idx]).reshape(EROWS, 128)
    dst2 = jnp.concatenate([dst, pad_idx]).reshape(EROWS, 128)
    ew_pad = jnp.concatenate([edge_weight, jnp.zeros((npad,), jnp.float32)])
    ew2 = ew_pad.reshape(EROWS, 128)

    x_pad = jnp.pad(x, ((0, N_PAD - N), (0, 0)))
    zeros1 = jnp.zeros((NODES_PER_TILE,), jnp.float32)
    zeros16 = jnp.zeros((NODES_PER_TILE, D_HID), jnp.float32)

    deg_parts = _sc_deg(dst2, ew2, zeros1).reshape(2, N_PAD, 1)
    h1p = _tc1(x_pad, W1, deg_parts)
    parts1 = _sc_edge(src2, dst2, ew2, h1p, zeros16)
    rp = _tc2(parts1, h1p, deg_parts, b1.reshape(1, D_HID))
    parts2 = _sc_edge(src2, dst2, ew2, rp, zeros16)
    out = _tc3(parts2, rp, deg_parts, W2, b2.reshape(1, D_OUT))
    return out[:N]


# speedup vs baseline: 47.5967x; 1.2074x over previous
import functools

import jax
import jax.numpy as jnp
from jax import lax
from jax.experimental import pallas as pl
from jax.experimental.pallas import tpu as pltpu
from jax.experimental.pallas import tpu_sc as plsc

N = 10000
E = 320000
D_IN = 128
D_HID = 16
D_OUT = 128

N_PAD = 10240
E_PAD = 327680
EROWS = E_PAD // 128
ROWS_PER_TILE = EROWS // 32
CHUNK_ROWS = 8
NCHUNK = ROWS_PER_TILE // CHUNK_ROWS
NODES_PER_TILE = N_PAD // 16

_MESH = plsc.VectorSubcoreMesh(core_axis_name="c", subcore_axis_name="s")


@functools.partial(
    pl.kernel,
    out_type=jax.ShapeDtypeStruct((2, N_PAD), jnp.float32),
    mesh=_MESH,
    scratch_types=[
        pltpu.VMEM_SHARED((N_PAD,), jnp.float32),
        pltpu.VMEM((CHUNK_ROWS, 128), jnp.int32),
        pltpu.VMEM((CHUNK_ROWS, 128), jnp.float32),
        pltpu.SemaphoreType.DMA,
    ],
    compiler_params=pltpu.CompilerParams(use_tc_tiling_on_sc=False, needs_layout_passes=False),
)
def _sc_deg(dst_hbm, ew_hbm, zeros_hbm, out_hbm, acc_sh, dstv, eww, sem):
    c = lax.axis_index("c")
    s = lax.axis_index("s")
    wid = c * 16 + s
    node0 = s * NODES_PER_TILE

    pltpu.sync_copy(zeros_hbm, acc_sh.at[pl.ds(node0, NODES_PER_TILE)])
    plsc.subcore_barrier()

    row0 = wid * ROWS_PER_TILE

    def chunk(ci, _):
        base = row0 + ci * CHUNK_ROWS
        pltpu.sync_copy(dst_hbm.at[pl.ds(base, CHUNK_ROWS)], dstv)
        pltpu.sync_copy(ew_hbm.at[pl.ds(base, CHUNK_ROWS)], eww)
        descs = []
        for j in range(CHUNK_ROWS):
            descs.append(
                pltpu.async_copy(eww.at[j], acc_sh.at[dstv.at[j]], sem, add=True)
            )
        for d in descs:
            d.wait()
        return ()

    lax.fori_loop(0, NCHUNK, chunk, (), unroll=False)
    plsc.subcore_barrier()
    pltpu.sync_copy(
        acc_sh.at[pl.ds(node0, NODES_PER_TILE)],
        out_hbm.at[c, pl.ds(node0, NODES_PER_TILE)],
    )


@functools.partial(
    pl.kernel,
    out_type=jax.ShapeDtypeStruct((2, N_PAD, D_HID), jnp.float32),
    mesh=_MESH,
    scratch_types=[
        pltpu.VMEM_SHARED((N_PAD, D_HID), jnp.float32),
        pltpu.VMEM((CHUNK_ROWS, 3, 128), jnp.int32),
        pltpu.VMEM((CHUNK_ROWS, 3, 128), jnp.int32),
        pltpu.VMEM((CHUNK_ROWS, 3, 128), jnp.int32),
        pltpu.VMEM((CHUNK_ROWS, 128, D_HID), jnp.float32),
        pltpu.VMEM((CHUNK_ROWS, 128, D_HID), jnp.float32),
        pltpu.VMEM((CHUNK_ROWS, 128, D_HID), jnp.float32),
        pltpu.SemaphoreType.DMA,
        pltpu.SemaphoreType.DMA,
        pltpu.SemaphoreType.DMA,
        pltpu.SemaphoreType.DMA,
        pltpu.SemaphoreType.DMA,
        pltpu.SemaphoreType.DMA,
    ],
    compiler_params=pltpu.CompilerParams(use_tc_tiling_on_sc=False, needs_layout_passes=False),
)
def _sc_edge(edges_hbm, feat_hbm, zeros_hbm, out_hbm,
             acc_sh, eb0, eb1, eb2, rows0, rows1, rows2,
             gs0, gs1, gs2, ss0, ss1, ss2):
    c = lax.axis_index("c")
    s = lax.axis_index("s")
    wid = c * 16 + s
    node0 = s * NODES_PER_TILE

    pltpu.sync_copy(zeros_hbm, acc_sh.at[pl.ds(node0, NODES_PER_TILE)])
    plsc.subcore_barrier()

    row0 = wid * ROWS_PER_TILE
    ebs = (eb0, eb1, eb2)
    rowss = (rows0, rows1, rows2)
    gss = (gs0, gs1, gs2)
    sss = (ss0, ss1, ss2)

    def idx_copy(ci):
        pltpu.sync_copy(
            edges_hbm.at[pl.ds(row0 + ci * CHUNK_ROWS, CHUNK_ROWS)], ebs[ci % 3])

    def fire_gathers(ci):
        eb, rows, sem = ebs[ci % 3], rowss[ci % 3], gss[ci % 3]
        for j in range(CHUNK_ROWS):
            pltpu.async_copy(feat_hbm.at[eb.at[j, 0]], rows.at[j], sem)

    def wait_gathers(ci):
        eb, rows, sem = ebs[ci % 3], rowss[ci % 3], gss[ci % 3]
        for j in range(CHUNK_ROWS):
            pltpu.make_async_copy(feat_hbm.at[eb.at[j, 0]], rows.at[j], sem).wait()

    def fire_scatters(ci):
        eb, rows, sem = ebs[ci % 3], rowss[ci % 3], sss[ci % 3]
        for j in range(CHUNK_ROWS):
            pltpu.async_copy(rows.at[j], acc_sh.at[eb.at[j, 1]], sem, add=True)

    def wait_scatters(ci):
        eb, rows, sem = ebs[ci % 3], rowss[ci % 3], sss[ci % 3]
        for j in range(CHUNK_ROWS):
            pltpu.make_async_copy(rows.at[j], acc_sh.at[eb.at[j, 1]], sem).wait()

    def scale(ci):
        eb, rows = ebs[ci % 3], rowss[ci % 3]

        def body(g, _):
            j = g >> 3
            e0 = (g & 7) * 16
            wv = plsc.bitcast(eb[j, 2, pl.ds(e0, 16)], jnp.float32)
            for k in range(16):
                rows[j, e0 + k, :] = rows[j, e0 + k, :] * wv[k]
            return ()

        lax.fori_loop(0, CHUNK_ROWS * 8, body, (), unroll=False)

    idx_copy(0)
    fire_gathers(0)
    for ci in range(NCHUNK):
        wait_gathers(ci)
        if ci >= 2:
            wait_scatters(ci - 2)
        if ci + 1 < NCHUNK:
            idx_copy(ci + 1)
            fire_gathers(ci + 1)
        scale(ci)
        fire_scatters(ci)
    wait_scatters(NCHUNK - 2)
    wait_scatters(NCHUNK - 1)

    plsc.subcore_barrier()
    pltpu.sync_copy(
        acc_sh.at[pl.ds(node0, NODES_PER_TILE)],
        out_hbm.at[c, pl.ds(node0, NODES_PER_TILE)],
    )


_BLK = 1024
_GRID = N_PAD // _BLK


def _tc1_body(x_ref, w_ref, deg_ref, o_ref):
    dis = lax.rsqrt(deg_ref[0] + deg_ref[1] + 1.0)
    h = jnp.dot(x_ref[...], w_ref[...], preferred_element_type=jnp.float32)
    o_ref[...] = h * dis


def _tc2_body(p_ref, h_ref, deg_ref, b_ref, o_ref):
    dis = lax.rsqrt(deg_ref[0] + deg_ref[1] + 1.0)
    agg = dis * (p_ref[0] + p_ref[1] + h_ref[...]) + b_ref[...]
    o_ref[...] = dis * jnp.maximum(agg, 0.0)


def _tc3_body(q_ref, r_ref, deg_ref, w_ref, b_ref, o_ref):
    dis = lax.rsqrt(deg_ref[0] + deg_ref[1] + 1.0)
    agg = dis * (q_ref[0] + q_ref[1] + r_ref[...])
    y = jnp.dot(agg, w_ref[...], preferred_element_type=jnp.float32) + b_ref[...]
    m = jnp.max(y, axis=1, keepdims=True)
    lse = m + jnp.log(jnp.sum(jnp.exp(y - m), axis=1, keepdims=True))
    o_ref[...] = y - lse


def _tc1(x_pad, W1, deg_parts):
    return pl.pallas_call(
        _tc1_body,
        grid=(_GRID,),
        in_specs=[
            pl.BlockSpec((_BLK, D_IN), lambda i: (i, 0)),
            pl.BlockSpec((D_IN, D_HID), lambda i: (0, 0)),
            pl.BlockSpec((2, _BLK, 1), lambda i: (0, i, 0)),
        ],
        out_specs=pl.BlockSpec((_BLK, D_HID), lambda i: (i, 0)),
        out_shape=jax.ShapeDtypeStruct((N_PAD, D_HID), jnp.float32),
    )(x_pad, W1, deg_parts)


def _tc2(parts, h1p, deg_parts, b1):
    return pl.pallas_call(
        _tc2_body,
        grid=(_GRID,),
        in_specs=[
            pl.BlockSpec((2, _BLK, D_HID), lambda i: (0, i, 0)),
            pl.BlockSpec((_BLK, D_HID), lambda i: (i, 0)),
            pl.BlockSpec((2, _BLK, 1), lambda i: (0, i, 0)),
            pl.BlockSpec((1, D_HID), lambda i: (0, 0)),
        ],
        out_specs=pl.BlockSpec((_BLK, D_HID), lambda i: (i, 0)),
        out_shape=jax.ShapeDtypeStruct((N_PAD, D_HID), jnp.float32),
    )(parts, h1p, deg_parts, b1)


def _tc3(parts, rp, deg_parts, W2, b2):
    return pl.pallas_call(
        _tc3_body,
        grid=(_GRID,),
        in_specs=[
            pl.BlockSpec((2, _BLK, D_HID), lambda i: (0, i, 0)),
            pl.BlockSpec((_BLK, D_HID), lambda i: (i, 0)),
            pl.BlockSpec((2, _BLK, 1), lambda i: (0, i, 0)),
            pl.BlockSpec((D_HID, D_OUT), lambda i: (0, 0)),
            pl.BlockSpec((1, D_OUT), lambda i: (0, 0)),
        ],
        out_specs=pl.BlockSpec((_BLK, D_OUT), lambda i: (i, 0)),
        out_shape=jax.ShapeDtypeStruct((N_PAD, D_OUT), jnp.float32),
    )(parts, rp, deg_parts, W2, b2)


def kernel(x, edge_index, edge_weight, W1, b1, W2, b2):
    src = edge_index[0]
    dst = edge_index[1]

    npad = E_PAD - E
    pad_idx = N + (jnp.arange(npad, dtype=jnp.int32) % (N_PAD - N))
    src2 = jnp.concatenate([src, pad_idx]).reshape(EROWS, 128)
    dst2 = jnp.concatenate([dst, pad_idx]).reshape(EROWS, 128)
    ew_pad = jnp.concatenate([edge_weight, jnp.zeros((npad,), jnp.float32)])
    ew2 = ew_pad.reshape(EROWS, 128)
    edges = jnp.stack(
        [src2, dst2, jax.lax.bitcast_convert_type(ew2, jnp.int32)], axis=1)

    x_pad = jnp.pad(x, ((0, N_PAD - N), (0, 0)))
    zeros1 = jnp.zeros((NODES_PER_TILE,), jnp.float32)
    zeros16 = jnp.zeros((NODES_PER_TILE, D_HID), jnp.float32)

    deg_parts = _sc_deg(dst2, ew2, zeros1).reshape(2, N_PAD, 1)
    h1p = _tc1(x_pad, W1, deg_parts)
    parts1 = _sc_edge(edges, h1p, zeros16)
    rp = _tc2(parts1, h1p, deg_parts, b1.reshape(1, D_HID))
    parts2 = _sc_edge(edges, rp, zeros16)
    out = _tc3(parts2, rp, deg_parts, W2, b2.reshape(1, D_OUT))
    return out[:N]


# speedup vs baseline: 59.5887x; 1.2519x over previous
import functools

import jax
import jax.numpy as jnp
from jax import lax
from jax.experimental import pallas as pl
from jax.experimental.pallas import tpu as pltpu
from jax.experimental.pallas import tpu_sc as plsc

N = 10000
E = 320000
D_IN = 128
D_HID = 16
D_OUT = 128

N_PAD = 10240
E_PAD = 327680
EROWS = E_PAD // 128
ROWS_PER_TILE = EROWS // 32
CHUNK_ROWS = 8
NCHUNK = ROWS_PER_TILE // CHUNK_ROWS
NODES_PER_TILE = N_PAD // 16

_MESH = plsc.VectorSubcoreMesh(core_axis_name="c", subcore_axis_name="s")


@functools.partial(
    pl.kernel,
    out_type=jax.ShapeDtypeStruct((2, N_PAD, D_HID), jnp.float32),
    mesh=_MESH,
    scratch_types=[
        pltpu.VMEM_SHARED((N_PAD, D_HID), jnp.float32),
        pltpu.VMEM((CHUNK_ROWS, 3, 128), jnp.int32),
        pltpu.VMEM((CHUNK_ROWS, 3, 128), jnp.int32),
        pltpu.VMEM((CHUNK_ROWS, 128, D_HID), jnp.float32),
        pltpu.VMEM((CHUNK_ROWS, 128, D_HID), jnp.float32),
        pltpu.SemaphoreType.DMA,
        pltpu.SemaphoreType.DMA,
    ],
    compiler_params=pltpu.CompilerParams(use_tc_tiling_on_sc=False, needs_layout_passes=False),
)
def _sc_deg(edges_hbm, zeros_hbm, out_hbm, acc_sh, eb0, eb1, rows0, rows1,
            ss0, ss1):
    c = lax.axis_index("c")
    s = lax.axis_index("s")
    wid = c * 16 + s
    node0 = s * NODES_PER_TILE

    pltpu.sync_copy(zeros_hbm, acc_sh.at[pl.ds(node0, NODES_PER_TILE)])
    plsc.subcore_barrier()

    row0 = wid * ROWS_PER_TILE
    ebs = (eb0, eb1)
    rowss = (rows0, rows1)
    sss = (ss0, ss1)

    def wait_scatters(ci):
        eb, rows, sem = ebs[ci % 2], rowss[ci % 2], sss[ci % 2]
        for j in range(CHUNK_ROWS):
            pltpu.make_async_copy(rows.at[j], acc_sh.at[eb.at[j, 1]], sem).wait()

    for ci in range(NCHUNK):
        if ci >= 2:
            wait_scatters(ci - 2)
        eb, rows, sem = ebs[ci % 2], rowss[ci % 2], sss[ci % 2]
        pltpu.sync_copy(
            edges_hbm.at[pl.ds(row0 + ci * CHUNK_ROWS, CHUNK_ROWS)], eb)

        def body(g, _):
            j = g >> 3
            e0 = (g & 7) * 16
            wv = plsc.bitcast(eb[j, 2, pl.ds(e0, 16)], jnp.float32)
            for k in range(16):
                rows[j, e0 + k, :] = jnp.broadcast_to(wv[k], (D_HID,))
            return ()

        lax.fori_loop(0, CHUNK_ROWS * 8, body, (), unroll=False)
        for j in range(CHUNK_ROWS):
            pltpu.async_copy(rows.at[j], acc_sh.at[eb.at[j, 1]], sem, add=True)
    wait_scatters(NCHUNK - 2)
    wait_scatters(NCHUNK - 1)

    plsc.subcore_barrier()
    pltpu.sync_copy(
        acc_sh.at[pl.ds(node0, NODES_PER_TILE)],
        out_hbm.at[c, pl.ds(node0, NODES_PER_TILE)],
    )


@functools.partial(
    pl.kernel,
    out_type=jax.ShapeDtypeStruct((2, N_PAD, D_HID), jnp.float32),
    mesh=_MESH,
    scratch_types=[
        pltpu.VMEM_SHARED((N_PAD, D_HID), jnp.float32),
        pltpu.VMEM((CHUNK_ROWS, 3, 128), jnp.int32),
        pltpu.VMEM((CHUNK_ROWS, 3, 128), jnp.int32),
        pltpu.VMEM((CHUNK_ROWS, 3, 128), jnp.int32),
        pltpu.VMEM((CHUNK_ROWS, 128, D_HID), jnp.float32),
        pltpu.VMEM((CHUNK_ROWS, 128, D_HID), jnp.float32),
        pltpu.VMEM((CHUNK_ROWS, 128, D_HID), jnp.float32),
        pltpu.SemaphoreType.DMA,
        pltpu.SemaphoreType.DMA,
        pltpu.SemaphoreType.DMA,
        pltpu.SemaphoreType.DMA,
        pltpu.SemaphoreType.DMA,
        pltpu.SemaphoreType.DMA,
    ],
    compiler_params=pltpu.CompilerParams(use_tc_tiling_on_sc=False, needs_layout_passes=False),
)
def _sc_edge(edges_hbm, feat_hbm, zeros_hbm, out_hbm,
             acc_sh, eb0, eb1, eb2, rows0, rows1, rows2,
             gs0, gs1, gs2, ss0, ss1, ss2):
    c = lax.axis_index("c")
    s = lax.axis_index("s")
    wid = c * 16 + s
    node0 = s * NODES_PER_TILE

    pltpu.sync_copy(zeros_hbm, acc_sh.at[pl.ds(node0, NODES_PER_TILE)])
    plsc.subcore_barrier()

    row0 = wid * ROWS_PER_TILE
    ebs = (eb0, eb1, eb2)
    rowss = (rows0, rows1, rows2)
    gss = (gs0, gs1, gs2)
    sss = (ss0, ss1, ss2)

    def idx_copy(ci):
        pltpu.sync_copy(
            edges_hbm.at[pl.ds(row0 + ci * CHUNK_ROWS, CHUNK_ROWS)], ebs[ci % 3])

    def fire_gathers(ci):
        eb, rows, sem = ebs[ci % 3], rowss[ci % 3], gss[ci % 3]
        for j in range(CHUNK_ROWS):
            pltpu.async_copy(feat_hbm.at[eb.at[j, 0]], rows.at[j], sem)

    def wait_gathers(ci):
        eb, rows, sem = ebs[ci % 3], rowss[ci % 3], gss[ci % 3]
        for j in range(CHUNK_ROWS):
            pltpu.make_async_copy(feat_hbm.at[eb.at[j, 0]], rows.at[j], sem).wait()

    def fire_scatters(ci):
        eb, rows, sem = ebs[ci % 3], rowss[ci % 3], sss[ci % 3]
        for j in range(CHUNK_ROWS):
            pltpu.async_copy(rows.at[j], acc_sh.at[eb.at[j, 1]], sem, add=True)

    def wait_scatters(ci):
        eb, rows, sem = ebs[ci % 3], rowss[ci % 3], sss[ci % 3]
        for j in range(CHUNK_ROWS):
            pltpu.make_async_copy(rows.at[j], acc_sh.at[eb.at[j, 1]], sem).wait()

    def scale(ci):
        eb, rows = ebs[ci % 3], rowss[ci % 3]

        def body(g, _):
            j = g >> 3
            e0 = (g & 7) * 16
            wv = plsc.bitcast(eb[j, 2, pl.ds(e0, 16)], jnp.float32)
            for k in range(16):
                rows[j, e0 + k, :] = rows[j, e0 + k, :] * wv[k]
            return ()

        lax.fori_loop(0, CHUNK_ROWS * 8, body, (), unroll=False)

    idx_copy(0)
    fire_gathers(0)
    for ci in range(NCHUNK):
        wait_gathers(ci)
        if ci >= 2:
            wait_scatters(ci - 2)
        if ci + 1 < NCHUNK:
            idx_copy(ci + 1)
            fire_gathers(ci + 1)
        scale(ci)
        fire_scatters(ci)
    wait_scatters(NCHUNK - 2)
    wait_scatters(NCHUNK - 1)

    plsc.subcore_barrier()
    pltpu.sync_copy(
        acc_sh.at[pl.ds(node0, NODES_PER_TILE)],
        out_hbm.at[c, pl.ds(node0, NODES_PER_TILE)],
    )


PROWS = N_PAD // 8
_BLK = 128
_GRID = PROWS // _BLK


def _tc1_body(x_ref, w_ref, deg_ref, o_ref):
    dis = lax.rsqrt(deg_ref[0] + deg_ref[1] + 1.0)
    xv = x_ref[...].reshape(_BLK, 8 * D_IN)
    h = jnp.dot(xv, w_ref[...], preferred_element_type=jnp.float32)
    o_ref[...] = h * dis


def _tc2_body(p_ref, h_ref, deg_ref, b_ref, o_ref):
    dis = lax.rsqrt(deg_ref[0] + deg_ref[1] + 1.0)
    agg = dis * (p_ref[0] + p_ref[1] + h_ref[...]) + b_ref[...]
    o_ref[...] = dis * jnp.maximum(agg, 0.0)


def _tc3_body(q_ref, r_ref, deg_ref, w_ref, b_ref, o_ref):
    dis = lax.rsqrt(deg_ref[0] + deg_ref[1] + 1.0)
    agg = dis * (q_ref[0] + q_ref[1] + r_ref[...])
    y = jnp.dot(agg, w_ref[...], preferred_element_type=jnp.float32) + b_ref[...]
    y = y.reshape(_BLK, 8, D_OUT)
    m = jnp.max(y, axis=2, keepdims=True)
    lse = m + jnp.log(jnp.sum(jnp.exp(y - m), axis=2, keepdims=True))
    o_ref[...] = y - lse


def _tc1(x3, W1b, deg_parts):
    return pl.pallas_call(
        _tc1_body,
        grid=(_GRID,),
        in_specs=[
            pl.BlockSpec((_BLK, 8, D_IN), lambda i: (i, 0, 0)),
            pl.BlockSpec((8 * D_IN, 128), lambda i: (0, 0)),
            pl.BlockSpec((2, _BLK, 128), lambda i: (0, i, 0)),
        ],
        out_specs=pl.BlockSpec((_BLK, 128), lambda i: (i, 0)),
        out_shape=jax.ShapeDtypeStruct((PROWS, 128), jnp.float32),
    )(x3, W1b, deg_parts)


def _tc2(parts, h1p, deg_parts, b1p):
    return pl.pallas_call(
        _tc2_body,
        grid=(_GRID,),
        in_specs=[
            pl.BlockSpec((2, _BLK, 128), lambda i: (0, i, 0)),
            pl.BlockSpec((_BLK, 128), lambda i: (i, 0)),
            pl.BlockSpec((2, _BLK, 128), lambda i: (0, i, 0)),
            pl.BlockSpec((1, 128), lambda i: (0, 0)),
        ],
        out_specs=pl.BlockSpec((_BLK, 128), lambda i: (i, 0)),
        out_shape=jax.ShapeDtypeStruct((PROWS, 128), jnp.float32),
    )(parts, h1p, deg_parts, b1p)


def _tc3(parts, rp, deg_parts, W2b, b2p):
    return pl.pallas_call(
        _tc3_body,
        grid=(_GRID,),
        in_specs=[
            pl.BlockSpec((2, _BLK, 128), lambda i: (0, i, 0)),
            pl.BlockSpec((_BLK, 128), lambda i: (i, 0)),
            pl.BlockSpec((2, _BLK, 128), lambda i: (0, i, 0)),
            pl.BlockSpec((128, 8 * D_OUT), lambda i: (0, 0)),
            pl.BlockSpec((1, 8 * D_OUT), lambda i: (0, 0)),
        ],
        out_specs=pl.BlockSpec((_BLK, 8, D_OUT), lambda i: (i, 0, 0)),
        out_shape=jax.ShapeDtypeStruct((PROWS, 8, D_OUT), jnp.float32),
    )(parts, rp, deg_parts, W2b, b2p)


def kernel(x, edge_index, edge_weight, W1, b1, W2, b2):
    src = edge_index[0]
    dst = edge_index[1]

    npad = E_PAD - E
    pad_idx = N + (jnp.arange(npad, dtype=jnp.int32) % (N_PAD - N))
    src2 = jnp.concatenate([src, pad_idx]).reshape(EROWS, 128)
    dst2 = jnp.concatenate([dst, pad_idx]).reshape(EROWS, 128)
    ew_pad = jnp.concatenate([edge_weight, jnp.zeros((npad,), jnp.float32)])
    ew2 = ew_pad.reshape(EROWS, 128)
    edges = jnp.stack(
        [src2, dst2, jax.lax.bitcast_convert_type(ew2, jnp.int32)], axis=1)

    x3 = jnp.pad(x, ((0, N_PAD - N), (0, 0))).reshape(PROWS, 8, D_IN)
    zeros16 = jnp.zeros((NODES_PER_TILE, D_HID), jnp.float32)
    W1b = jnp.kron(jnp.eye(8, dtype=jnp.float32), W1)
    W2b = jnp.kron(jnp.eye(8, dtype=jnp.float32), W2)
    b1p = jnp.tile(b1, 8).reshape(1, 128)
    b2p = jnp.tile(b2, 8).reshape(1, 8 * D_OUT)

    deg_parts = _sc_deg(edges, zeros16).reshape(2, PROWS, 128)
    h1p = _tc1(x3, W1b, deg_parts)
    parts1 = _sc_edge(edges, h1p.reshape(N_PAD, D_HID), zeros16)
    rp = _tc2(parts1.reshape(2, PROWS, 128), h1p, deg_parts, b1p)
    parts2 = _sc_edge(edges, rp.reshape(N_PAD, D_HID), zeros16)
    out = _tc3(parts2.reshape(2, PROWS, 128), rp, deg_parts, W2b, b2p)
    return out.reshape(N_PAD, D_OUT)[:N]


# speedup vs baseline: 59.8419x; 1.0042x over previous
import functools

import jax
import jax.numpy as jnp
from jax import lax
from jax.experimental import pallas as pl
from jax.experimental.pallas import tpu as pltpu
from jax.experimental.pallas import tpu_sc as plsc

N = 10000
E = 320000
D_IN = 128
D_HID = 16
D_OUT = 128

N_PAD = 10240
E_PAD = 327680
EROWS = E_PAD // 128
ROWS_PER_TILE = EROWS // 32
CHUNK_ROWS = 8
NCHUNK = ROWS_PER_TILE // CHUNK_ROWS
NODES_PER_TILE = N_PAD // 16

_MESH = plsc.VectorSubcoreMesh(core_axis_name="c", subcore_axis_name="s")


@functools.partial(
    pl.kernel,
    out_type=jax.ShapeDtypeStruct((2, N_PAD, D_HID), jnp.float32),
    mesh=_MESH,
    scratch_types=[
        pltpu.VMEM_SHARED((N_PAD, D_HID), jnp.float32),
        pltpu.VMEM((CHUNK_ROWS, 3, 128), jnp.int32),
        pltpu.VMEM((CHUNK_ROWS, 3, 128), jnp.int32),
        pltpu.VMEM((CHUNK_ROWS, 128, D_HID), jnp.float32),
        pltpu.VMEM((CHUNK_ROWS, 128, D_HID), jnp.float32),
        pltpu.SemaphoreType.DMA,
        pltpu.SemaphoreType.DMA,
    ],
    compiler_params=pltpu.CompilerParams(use_tc_tiling_on_sc=False, needs_layout_passes=False),
)
def _sc_deg(edges_hbm, zeros_hbm, out_hbm, acc_sh, eb0, eb1, rows0, rows1,
            ss0, ss1):
    c = lax.axis_index("c")
    s = lax.axis_index("s")
    wid = c * 16 + s
    node0 = s * NODES_PER_TILE

    pltpu.sync_copy(zeros_hbm, acc_sh.at[pl.ds(node0, NODES_PER_TILE)])
    plsc.subcore_barrier()

    row0 = wid * ROWS_PER_TILE
    ebs = (eb0, eb1)
    rowss = (rows0, rows1)
    sss = (ss0, ss1)

    def wait_scatters(ci):
        eb, rows, sem = ebs[ci % 2], rowss[ci % 2], sss[ci % 2]
        for j in range(CHUNK_ROWS):
            pltpu.make_async_copy(rows.at[j], acc_sh.at[eb.at[j, 1]], sem).wait()

    for ci in range(NCHUNK):
        if ci >= 2:
            wait_scatters(ci - 2)
        eb, rows, sem = ebs[ci % 2], rowss[ci % 2], sss[ci % 2]
        pltpu.sync_copy(
            edges_hbm.at[pl.ds(row0 + ci * CHUNK_ROWS, CHUNK_ROWS)], eb)

        @plsc.parallel_loop(0, CHUNK_ROWS * 8, unroll=2)
        def _(g):
            j = g >> 3
            e0 = (g & 7) * 16
            wv = plsc.bitcast(eb[j, 2, pl.ds(e0, 16)], jnp.float32)
            for k in range(16):
                rows[j, e0 + k, :] = jnp.broadcast_to(wv[k], (D_HID,))
        for j in range(CHUNK_ROWS):
            pltpu.async_copy(rows.at[j], acc_sh.at[eb.at[j, 1]], sem, add=True)
    wait_scatters(NCHUNK - 2)
    wait_scatters(NCHUNK - 1)

    plsc.subcore_barrier()
    pltpu.sync_copy(
        acc_sh.at[pl.ds(node0, NODES_PER_TILE)],
        out_hbm.at[c, pl.ds(node0, NODES_PER_TILE)],
    )


@functools.partial(
    pl.kernel,
    out_type=jax.ShapeDtypeStruct((2, N_PAD, D_HID), jnp.float32),
    mesh=_MESH,
    scratch_types=[
        pltpu.VMEM_SHARED((N_PAD, D_HID), jnp.float32),
        pltpu.VMEM((CHUNK_ROWS, 3, 128), jnp.int32),
        pltpu.VMEM((CHUNK_ROWS, 3, 128), jnp.int32),
        pltpu.VMEM((CHUNK_ROWS, 3, 128), jnp.int32),
        pltpu.VMEM((CHUNK_ROWS, 128, D_HID), jnp.float32),
        pltpu.VMEM((CHUNK_ROWS, 128, D_HID), jnp.float32),
        pltpu.VMEM((CHUNK_ROWS, 128, D_HID), jnp.float32),
        pltpu.SemaphoreType.DMA,
        pltpu.SemaphoreType.DMA,
        pltpu.SemaphoreType.DMA,
        pltpu.SemaphoreType.DMA,
        pltpu.SemaphoreType.DMA,
        pltpu.SemaphoreType.DMA,
    ],
    compiler_params=pltpu.CompilerParams(use_tc_tiling_on_sc=False, needs_layout_passes=False),
)
def _sc_edge(edges_hbm, feat_hbm, zeros_hbm, out_hbm,
             acc_sh, eb0, eb1, eb2, rows0, rows1, rows2,
             gs0, gs1, gs2, ss0, ss1, ss2):
    c = lax.axis_index("c")
    s = lax.axis_index("s")
    wid = c * 16 + s
    node0 = s * NODES_PER_TILE

    pltpu.sync_copy(zeros_hbm, acc_sh.at[pl.ds(node0, NODES_PER_TILE)])
    plsc.subcore_barrier()

    row0 = wid * ROWS_PER_TILE
    ebs = (eb0, eb1, eb2)
    rowss = (rows0, rows1, rows2)
    gss = (gs0, gs1, gs2)
    sss = (ss0, ss1, ss2)

    def idx_copy(ci):
        pltpu.sync_copy(
            edges_hbm.at[pl.ds(row0 + ci * CHUNK_ROWS, CHUNK_ROWS)], ebs[ci % 3])

    def fire_gathers(ci):
        eb, rows, sem = ebs[ci % 3], rowss[ci % 3], gss[ci % 3]
        for j in range(CHUNK_ROWS):
            pltpu.async_copy(feat_hbm.at[eb.at[j, 0]], rows.at[j], sem)

    def wait_gathers(ci):
        eb, rows, sem = ebs[ci % 3], rowss[ci % 3], gss[ci % 3]
        for j in range(CHUNK_ROWS):
            pltpu.make_async_copy(feat_hbm.at[eb.at[j, 0]], rows.at[j], sem).wait()

    def fire_scatters(ci):
        eb, rows, sem = ebs[ci % 3], rowss[ci % 3], sss[ci % 3]
        for j in range(CHUNK_ROWS):
            pltpu.async_copy(rows.at[j], acc_sh.at[eb.at[j, 1]], sem, add=True)

    def wait_scatters(ci):
        eb, rows, sem = ebs[ci % 3], rowss[ci % 3], sss[ci % 3]
        for j in range(CHUNK_ROWS):
            pltpu.make_async_copy(rows.at[j], acc_sh.at[eb.at[j, 1]], sem).wait()

    def scale(ci):
        eb, rows = ebs[ci % 3], rowss[ci % 3]

        @plsc.parallel_loop(0, CHUNK_ROWS * 8, unroll=2)
        def _(g):
            j = g >> 3
            e0 = (g & 7) * 16
            wv = plsc.bitcast(eb[j, 2, pl.ds(e0, 16)], jnp.float32)
            for k in range(16):
                rows[j, e0 + k, :] = rows[j, e0 + k, :] * wv[k]

    idx_copy(0)
    fire_gathers(0)
    for ci in range(NCHUNK):
        wait_gathers(ci)
        if ci >= 2:
            wait_scatters(ci - 2)
        if ci + 1 < NCHUNK:
            idx_copy(ci + 1)
            fire_gathers(ci + 1)
        scale(ci)
        fire_scatters(ci)
    wait_scatters(NCHUNK - 2)
    wait_scatters(NCHUNK - 1)

    plsc.subcore_barrier()
    pltpu.sync_copy(
        acc_sh.at[pl.ds(node0, NODES_PER_TILE)],
        out_hbm.at[c, pl.ds(node0, NODES_PER_TILE)],
    )


PROWS = N_PAD // 8
_BLK = 128
_GRID = PROWS // _BLK


def _tc1_body(x_ref, w_ref, deg_ref, o_ref):
    dis = lax.rsqrt(deg_ref[0] + deg_ref[1] + 1.0)
    xv = x_ref[...].reshape(_BLK, 8 * D_IN)
    h = jnp.dot(xv, w_ref[...], preferred_element_type=jnp.float32)
    o_ref[...] = h * dis


def _tc2_body(p_ref, h_ref, deg_ref, b_ref, o_ref):
    dis = lax.rsqrt(deg_ref[0] + deg_ref[1] + 1.0)
    agg = dis * (p_ref[0] + p_ref[1] + h_ref[...]) + b_ref[...]
    o_ref[...] = dis * jnp.maximum(agg, 0.0)


def _tc3_body(q_ref, r_ref, deg_ref, w_ref, b_ref, o_ref):
    dis = lax.rsqrt(deg_ref[0] + deg_ref[1] + 1.0)
    agg = dis * (q_ref[0] + q_ref[1] + r_ref[...])
    y = jnp.dot(agg, w_ref[...], preferred_element_type=jnp.float32) + b_ref[...]
    y = y.reshape(_BLK, 8, D_OUT)
    m = jnp.max(y, axis=2, keepdims=True)
    lse = m + jnp.log(jnp.sum(jnp.exp(y - m), axis=2, keepdims=True))
    o_ref[...] = y - lse


def _tc1(x3, W1b, deg_parts):
    return pl.pallas_call(
        _tc1_body,
        grid=(_GRID,),
        in_specs=[
            pl.BlockSpec((_BLK, 8, D_IN), lambda i: (i, 0, 0)),
            pl.BlockSpec((8 * D_IN, 128), lambda i: (0, 0)),
            pl.BlockSpec((2, _BLK, 128), lambda i: (0, i, 0)),
        ],
        out_specs=pl.BlockSpec((_BLK, 128), lambda i: (i, 0)),
        out_shape=jax.ShapeDtypeStruct((PROWS, 128), jnp.float32),
    )(x3, W1b, deg_parts)


def _tc2(parts, h1p, deg_parts, b1p):
    return pl.pallas_call(
        _tc2_body,
        grid=(_GRID,),
        in_specs=[
            pl.BlockSpec((2, _BLK, 128), lambda i: (0, i, 0)),
            pl.BlockSpec((_BLK, 128), lambda i: (i, 0)),
            pl.BlockSpec((2, _BLK, 128), lambda i: (0, i, 0)),
            pl.BlockSpec((1, 128), lambda i: (0, 0)),
        ],
        out_specs=pl.BlockSpec((_BLK, 128), lambda i: (i, 0)),
        out_shape=jax.ShapeDtypeStruct((PROWS, 128), jnp.float32),
    )(parts, h1p, deg_parts, b1p)


def _tc3(parts, rp, deg_parts, W2b, b2p):
    return pl.pallas_call(
        _tc3_body,
        grid=(_GRID,),
        in_specs=[
            pl.BlockSpec((2, _BLK, 128), lambda i: (0, i, 0)),
            pl.BlockSpec((_BLK, 128), lambda i: (i, 0)),
            pl.BlockSpec((2, _BLK, 128), lambda i: (0, i, 0)),
            pl.BlockSpec((128, 8 * D_OUT), lambda i: (0, 0)),
            pl.BlockSpec((1, 8 * D_OUT), lambda i: (0, 0)),
        ],
        out_specs=pl.BlockSpec((_BLK, 8, D_OUT), lambda i: (i, 0, 0)),
        out_shape=jax.ShapeDtypeStruct((PROWS, 8, D_OUT), jnp.float32),
    )(parts, rp, deg_parts, W2b, b2p)


def kernel(x, edge_index, edge_weight, W1, b1, W2, b2):
    src = edge_index[0]
    dst = edge_index[1]

    npad = E_PAD - E
    pad_idx = N + (jnp.arange(npad, dtype=jnp.int32) % (N_PAD - N))
    src2 = jnp.concatenate([src, pad_idx]).reshape(EROWS, 128)
    dst2 = jnp.concatenate([dst, pad_idx]).reshape(EROWS, 128)
    ew_pad = jnp.concatenate([edge_weight, jnp.zeros((npad,), jnp.float32)])
    ew2 = ew_pad.reshape(EROWS, 128)
    edges = jnp.stack(
        [src2, dst2, jax.lax.bitcast_convert_type(ew2, jnp.int32)], axis=1)

    x3 = jnp.pad(x, ((0, N_PAD - N), (0, 0))).reshape(PROWS, 8, D_IN)
    zeros16 = jnp.zeros((NODES_PER_TILE, D_HID), jnp.float32)
    W1b = jnp.kron(jnp.eye(8, dtype=jnp.float32), W1)
    W2b = jnp.kron(jnp.eye(8, dtype=jnp.float32), W2)
    b1p = jnp.tile(b1, 8).reshape(1, 128)
    b2p = jnp.tile(b2, 8).reshape(1, 8 * D_OUT)

    deg_parts = _sc_deg(edges, zeros16).reshape(2, PROWS, 128)
    h1p = _tc1(x3, W1b, deg_parts)
    parts1 = _sc_edge(edges, h1p.reshape(N_PAD, D_HID), zeros16)
    rp = _tc2(parts1.reshape(2, PROWS, 128), h1p, deg_parts, b1p)
    parts2 = _sc_edge(edges, rp.reshape(N_PAD, D_HID), zeros16)
    out = _tc3(parts2.reshape(2, PROWS, 128), rp, deg_parts, W2b, b2p)
    return out.reshape(N_PAD, D_OUT)[:N]


# speedup vs baseline: 65.9954x; 1.1028x over previous
import functools

import jax
import jax.numpy as jnp
from jax import lax
from jax.experimental import pallas as pl
from jax.experimental.pallas import tpu as pltpu
from jax.experimental.pallas import tpu_sc as plsc

N = 10000
E = 320000
D_IN = 128
D_HID = 16
D_OUT = 128

N_PAD = 10240
E_PAD = 327680
EROWS = E_PAD // 128
ROWS_PER_TILE = EROWS // 32
CHUNK_ROWS = 8
NCHUNK = ROWS_PER_TILE // CHUNK_ROWS
NODES_PER_TILE = N_PAD // 16

_MESH = plsc.VectorSubcoreMesh(core_axis_name="c", subcore_axis_name="s")


@functools.partial(
    pl.kernel,
    out_type=jax.ShapeDtypeStruct((2, N_PAD, D_HID), jnp.float32),
    mesh=_MESH,
    scratch_types=[
        pltpu.VMEM_SHARED((N_PAD, D_HID), jnp.float32),
        pltpu.VMEM((CHUNK_ROWS, 3, 128), jnp.int32),
        pltpu.VMEM((CHUNK_ROWS, 3, 128), jnp.int32),
        pltpu.VMEM((CHUNK_ROWS, 128, D_HID), jnp.float32),
        pltpu.VMEM((CHUNK_ROWS, 128, D_HID), jnp.float32),
        pltpu.SemaphoreType.DMA,
        pltpu.SemaphoreType.DMA,
    ],
    compiler_params=pltpu.CompilerParams(use_tc_tiling_on_sc=False, needs_layout_passes=False),
)
def _sc_deg(edges_hbm, zeros_hbm, out_hbm, acc_sh, eb0, eb1, rows0, rows1,
            ss0, ss1):
    c = lax.axis_index("c")
    s = lax.axis_index("s")
    wid = c * 16 + s
    node0 = s * NODES_PER_TILE

    pltpu.sync_copy(zeros_hbm, acc_sh.at[pl.ds(node0, NODES_PER_TILE)])
    plsc.subcore_barrier()

    row0 = wid * ROWS_PER_TILE
    ebs = (eb0, eb1)
    rowss = (rows0, rows1)
    sss = (ss0, ss1)

    def wait_scatters(ci):
        eb, rows, sem = ebs[ci % 2], rowss[ci % 2], sss[ci % 2]
        for j in range(CHUNK_ROWS):
            pltpu.make_async_copy(rows.at[j], acc_sh.at[eb.at[j, 1]], sem).wait()

    for ci in range(NCHUNK):
        if ci >= 2:
            wait_scatters(ci - 2)
        eb, rows, sem = ebs[ci % 2], rowss[ci % 2], sss[ci % 2]
        pltpu.sync_copy(
            edges_hbm.at[pl.ds(row0 + ci * CHUNK_ROWS, CHUNK_ROWS)], eb)

        @plsc.parallel_loop(0, CHUNK_ROWS * 8, unroll=2)
        def _(g):
            j = g >> 3
            e0 = (g & 7) * 16
            wv = plsc.bitcast(eb[j, 2, pl.ds(e0, 16)], jnp.float32)
            for k in range(16):
                rows[j, e0 + k, :] = jnp.broadcast_to(wv[k], (D_HID,))
        for j in range(CHUNK_ROWS):
            pltpu.async_copy(rows.at[j], acc_sh.at[eb.at[j, 1]], sem, add=True)
    wait_scatters(NCHUNK - 2)
    wait_scatters(NCHUNK - 1)

    plsc.subcore_barrier()
    pltpu.sync_copy(
        acc_sh.at[pl.ds(node0, NODES_PER_TILE)],
        out_hbm.at[c, pl.ds(node0, NODES_PER_TILE)],
    )


@functools.partial(
    pl.kernel,
    out_type=jax.ShapeDtypeStruct((2, N_PAD, D_HID), jnp.float32),
    mesh=_MESH,
    scratch_types=[
        pltpu.VMEM_SHARED((N_PAD, D_HID), jnp.float32),
        pltpu.VMEM((CHUNK_ROWS, 3, 128), jnp.int32),
        pltpu.VMEM((CHUNK_ROWS, 3, 128), jnp.int32),
        pltpu.VMEM((CHUNK_ROWS, 3, 128), jnp.int32),
        pltpu.VMEM((CHUNK_ROWS, 128, D_HID), jnp.float32),
        pltpu.VMEM((CHUNK_ROWS, 128, D_HID), jnp.float32),
        pltpu.VMEM((CHUNK_ROWS, 128, D_HID), jnp.float32),
        pltpu.SemaphoreType.DMA,
        pltpu.SemaphoreType.DMA,
        pltpu.SemaphoreType.DMA,
        pltpu.SemaphoreType.DMA,
        pltpu.SemaphoreType.DMA,
        pltpu.SemaphoreType.DMA,
    ],
    compiler_params=pltpu.CompilerParams(use_tc_tiling_on_sc=False, needs_layout_passes=False),
)
def _sc_edge(edges_hbm, feat_hbm, zeros_hbm, out_hbm,
             acc_sh, eb0, eb1, eb2, rows0, rows1, rows2,
             gs0, gs1, gs2, ss0, ss1, ss2):
    c = lax.axis_index("c")
    s = lax.axis_index("s")
    wid = c * 16 + s
    node0 = s * NODES_PER_TILE

    pltpu.sync_copy(zeros_hbm, acc_sh.at[pl.ds(node0, NODES_PER_TILE)])
    plsc.subcore_barrier()

    row0 = wid * ROWS_PER_TILE
    ebs = (eb0, eb1, eb2)
    rowss = (rows0, rows1, rows2)
    gss = (gs0, gs1, gs2)
    sss = (ss0, ss1, ss2)

    def idx_copy(ci):
        pltpu.sync_copy(
            edges_hbm.at[pl.ds(row0 + ci * CHUNK_ROWS, CHUNK_ROWS)], ebs[ci % 3])

    def fire_gathers(ci):
        eb, rows, sem = ebs[ci % 3], rowss[ci % 3], gss[ci % 3]
        for j in range(CHUNK_ROWS):
            pltpu.async_copy(feat_hbm.at[eb.at[j, 0]], rows.at[j], sem)

    def wait_gathers(ci):
        eb, rows, sem = ebs[ci % 3], rowss[ci % 3], gss[ci % 3]
        for j in range(CHUNK_ROWS):
            pltpu.make_async_copy(feat_hbm.at[eb.at[j, 0]], rows.at[j], sem).wait()

    def fire_scatters(ci):
        eb, rows, sem = ebs[ci % 3], rowss[ci % 3], sss[ci % 3]
        for j in range(CHUNK_ROWS):
            pltpu.async_copy(rows.at[j], acc_sh.at[eb.at[j, 1]], sem, add=True)

    def wait_scatters(ci):
        eb, rows, sem = ebs[ci % 3], rowss[ci % 3], sss[ci % 3]
        for j in range(CHUNK_ROWS):
            pltpu.make_async_copy(rows.at[j], acc_sh.at[eb.at[j, 1]], sem).wait()

    def scale(ci):
        eb, rows = ebs[ci % 3], rowss[ci % 3]

        @plsc.parallel_loop(0, CHUNK_ROWS * 8, unroll=2)
        def _(g):
            j = g >> 3
            e0 = (g & 7) * 16
            wv = plsc.bitcast(eb[j, 2, pl.ds(e0, 16)], jnp.float32)
            for k in range(16):
                rows[j, e0 + k, :] = rows[j, e0 + k, :] * wv[k]

    idx_copy(0)
    fire_gathers(0)
    for ci in range(NCHUNK):
        wait_gathers(ci)
        if ci >= 2:
            wait_scatters(ci - 2)
        if ci + 1 < NCHUNK:
            idx_copy(ci + 1)
            fire_gathers(ci + 1)
        scale(ci)
        fire_scatters(ci)
    wait_scatters(NCHUNK - 2)
    wait_scatters(NCHUNK - 1)

    plsc.subcore_barrier()
    pltpu.sync_copy(
        acc_sh.at[pl.ds(node0, NODES_PER_TILE)],
        out_hbm.at[c, pl.ds(node0, NODES_PER_TILE)],
    )


PROWS = N_PAD // 8
_BLK = 128
_GRID = PROWS // _BLK


def _tc1_body(x_ref, w_ref, deg_ref, o_ref):
    dis = lax.rsqrt(deg_ref[0] + deg_ref[1] + 1.0)
    xp = jnp.concatenate(
        [x_ref[...], jnp.zeros((N_PAD - N, D_IN), jnp.float32)], axis=0)
    xv = xp.reshape(PROWS, 8 * D_IN)
    h = jnp.dot(xv, w_ref[...], preferred_element_type=jnp.float32)
    o_ref[...] = h * dis


def _tc2_body(p_ref, h_ref, deg_ref, b_ref, o_ref):
    dis = lax.rsqrt(deg_ref[0] + deg_ref[1] + 1.0)
    agg = dis * (p_ref[0] + p_ref[1] + h_ref[...]) + b_ref[...]
    o_ref[...] = dis * jnp.maximum(agg, 0.0)


def _tc3_body(q_ref, r_ref, deg_ref, w_ref, b_ref, o_ref):
    dis = lax.rsqrt(deg_ref[0] + deg_ref[1] + 1.0)
    agg = dis * (q_ref[0] + q_ref[1] + r_ref[...])
    y = jnp.dot(agg, w_ref[...], preferred_element_type=jnp.float32) + b_ref[...]
    y = y.reshape(PROWS, 8, D_OUT)
    m = jnp.max(y, axis=2, keepdims=True)
    lse = m + jnp.log(jnp.sum(jnp.exp(y - m), axis=2, keepdims=True))
    o_ref[...] = (y - lse).reshape(N_PAD, D_OUT)[:N]


def _tc1(x, W1b, deg_parts):
    return pl.pallas_call(
        _tc1_body,
        out_shape=jax.ShapeDtypeStruct((PROWS, 128), jnp.float32),
    )(x, W1b, deg_parts)


def _tc2(parts, h1p, deg_parts, b1p):
    return pl.pallas_call(
        _tc2_body,
        out_shape=jax.ShapeDtypeStruct((PROWS, 128), jnp.float32),
    )(parts, h1p, deg_parts, b1p)


def _tc3(parts, rp, deg_parts, W2b, b2p):
    return pl.pallas_call(
        _tc3_body,
        out_shape=jax.ShapeDtypeStruct((N, D_OUT), jnp.float32),
    )(parts, rp, deg_parts, W2b, b2p)


def kernel(x, edge_index, edge_weight, W1, b1, W2, b2):
    src = edge_index[0]
    dst = edge_index[1]

    npad = E_PAD - E
    pad_idx = N + (jnp.arange(npad, dtype=jnp.int32) % (N_PAD - N))
    src2 = jnp.concatenate([src, pad_idx]).reshape(EROWS, 128)
    dst2 = jnp.concatenate([dst, pad_idx]).reshape(EROWS, 128)
    ew_pad = jnp.concatenate([edge_weight, jnp.zeros((npad,), jnp.float32)])
    ew2 = ew_pad.reshape(EROWS, 128)
    edges = jnp.stack(
        [src2, dst2, jax.lax.bitcast_convert_type(ew2, jnp.int32)], axis=1)

    zeros16 = jnp.zeros((NODES_PER_TILE, D_HID), jnp.float32)
    W1b = jnp.kron(jnp.eye(8, dtype=jnp.float32), W1)
    W2b = jnp.kron(jnp.eye(8, dtype=jnp.float32), W2)
    b1p = jnp.tile(b1, 8).reshape(1, 128)
    b2p = jnp.tile(b2, 8).reshape(1, 8 * D_OUT)

    deg_parts = _sc_deg(edges, zeros16).reshape(2, PROWS, 128)
    h1p = _tc1(x, W1b, deg_parts)
    parts1 = _sc_edge(edges, h1p.reshape(N_PAD, D_HID), zeros16)
    rp = _tc2(parts1.reshape(2, PROWS, 128), h1p, deg_parts, b1p)
    parts2 = _sc_edge(edges, rp.reshape(N_PAD, D_HID), zeros16)
    return _tc3(parts2.reshape(2, PROWS, 128), rp, deg_parts, W2b, b2p)


# speedup vs baseline: 71.2791x; 1.0801x over previous
import functools

import jax
import jax.numpy as jnp
from jax import lax
from jax.experimental import pallas as pl
from jax.experimental.pallas import tpu as pltpu
from jax.experimental.pallas import tpu_sc as plsc

N = 10000
E = 320000
D_IN = 128
D_HID = 16
D_OUT = 128

N_PAD = 10240
E_PAD = 327680
EROWS = E_PAD // 128
ROWS_PER_TILE = EROWS // 32
CHUNK_ROWS = 8
NCHUNK = ROWS_PER_TILE // CHUNK_ROWS
NODES_PER_TILE = N_PAD // 16

_MESH = plsc.VectorSubcoreMesh(core_axis_name="c", subcore_axis_name="s")


@functools.partial(
    pl.kernel,
    out_type=jax.ShapeDtypeStruct((2, N_PAD, D_HID), jnp.float32),
    mesh=_MESH,
    scratch_types=[
        pltpu.VMEM_SHARED((N_PAD, D_HID), jnp.float32),
        pltpu.VMEM((CHUNK_ROWS, 3, 128), jnp.int32),
        pltpu.VMEM((CHUNK_ROWS, 3, 128), jnp.int32),
        pltpu.VMEM((CHUNK_ROWS, 128, D_HID), jnp.float32),
        pltpu.VMEM((CHUNK_ROWS, 128, D_HID), jnp.float32),
        pltpu.SemaphoreType.DMA,
        pltpu.SemaphoreType.DMA,
    ],
    compiler_params=pltpu.CompilerParams(use_tc_tiling_on_sc=False, needs_layout_passes=False),
)
def _sc_deg(edges_hbm, zeros_hbm, out_hbm, acc_sh, eb0, eb1, rows0, rows1,
            ss0, ss1):
    c = lax.axis_index("c")
    s = lax.axis_index("s")
    wid = c * 16 + s
    node0 = s * NODES_PER_TILE

    pltpu.sync_copy(zeros_hbm, acc_sh.at[pl.ds(node0, NODES_PER_TILE)])
    plsc.subcore_barrier()

    row0 = wid * ROWS_PER_TILE
    ebs = (eb0, eb1)
    rowss = (rows0, rows1)
    sss = (ss0, ss1)

    def wait_scatters(ci):
        eb, rows, sem = ebs[ci % 2], rowss[ci % 2], sss[ci % 2]
        for j in range(CHUNK_ROWS):
            pltpu.make_async_copy(rows.at[j], acc_sh.at[eb.at[j, 1]], sem).wait()

    for ci in range(NCHUNK):
        if ci >= 2:
            wait_scatters(ci - 2)
        eb, rows, sem = ebs[ci % 2], rowss[ci % 2], sss[ci % 2]
        pltpu.sync_copy(
            edges_hbm.at[pl.ds(row0 + ci * CHUNK_ROWS, CHUNK_ROWS)], eb)

        @plsc.parallel_loop(0, CHUNK_ROWS * 8, unroll=2)
        def _(g):
            j = g >> 3
            e0 = (g & 7) * 16
            wv = plsc.bitcast(eb[j, 2, pl.ds(e0, 16)], jnp.float32)
            for k in range(16):
                rows[j, e0 + k, :] = jnp.broadcast_to(wv[k], (D_HID,))
        for j in range(CHUNK_ROWS):
            pltpu.async_copy(rows.at[j], acc_sh.at[eb.at[j, 1]], sem, add=True)
    wait_scatters(NCHUNK - 2)
    wait_scatters(NCHUNK - 1)

    plsc.subcore_barrier()
    pltpu.sync_copy(
        acc_sh.at[pl.ds(node0, NODES_PER_TILE)],
        out_hbm.at[c, pl.ds(node0, NODES_PER_TILE)],
    )


@functools.partial(
    pl.kernel,
    out_type=jax.ShapeDtypeStruct((2, N_PAD, D_HID), jnp.float32),
    mesh=_MESH,
    scratch_types=[
        pltpu.VMEM_SHARED((N_PAD, D_HID), jnp.float32),
        pltpu.VMEM_SHARED((N_PAD, D_HID), jnp.float32),
        pltpu.VMEM((CHUNK_ROWS, 3, 128), jnp.int32),
        pltpu.VMEM((CHUNK_ROWS, 3, 128), jnp.int32),
        pltpu.VMEM((CHUNK_ROWS, 3, 128), jnp.int32),
        pltpu.VMEM((CHUNK_ROWS, 128, D_HID), jnp.float32),
        pltpu.VMEM((CHUNK_ROWS, 128, D_HID), jnp.float32),
        pltpu.VMEM((CHUNK_ROWS, 128, D_HID), jnp.float32),
        pltpu.SemaphoreType.DMA,
        pltpu.SemaphoreType.DMA,
        pltpu.SemaphoreType.DMA,
        pltpu.SemaphoreType.DMA,
        pltpu.SemaphoreType.DMA,
        pltpu.SemaphoreType.DMA,
    ],
    compiler_params=pltpu.CompilerParams(use_tc_tiling_on_sc=False, needs_layout_passes=False),
)
def _sc_edge(edges_hbm, feat_hbm, zeros_hbm, out_hbm,
             acc_sh, feat_sh, eb0, eb1, eb2, rows0, rows1, rows2,
             gs0, gs1, gs2, ss0, ss1, ss2):
    c = lax.axis_index("c")
    s = lax.axis_index("s")
    wid = c * 16 + s
    node0 = s * NODES_PER_TILE

    pltpu.sync_copy(zeros_hbm, acc_sh.at[pl.ds(node0, NODES_PER_TILE)])
    pltpu.sync_copy(feat_hbm.at[pl.ds(node0, NODES_PER_TILE)],
                    feat_sh.at[pl.ds(node0, NODES_PER_TILE)])
    plsc.subcore_barrier()

    row0 = wid * ROWS_PER_TILE
    ebs = (eb0, eb1, eb2)
    rowss = (rows0, rows1, rows2)
    gss = (gs0, gs1, gs2)
    sss = (ss0, ss1, ss2)

    def idx_copy(ci):
        pltpu.sync_copy(
            edges_hbm.at[pl.ds(row0 + ci * CHUNK_ROWS, CHUNK_ROWS)], ebs[ci % 3])

    def fire_gathers(ci):
        eb, rows, sem = ebs[ci % 3], rowss[ci % 3], gss[ci % 3]
        for j in range(CHUNK_ROWS):
            pltpu.async_copy(feat_sh.at[eb.at[j, 0]], rows.at[j], sem)

    def wait_gathers(ci):
        eb, rows, sem = ebs[ci % 3], rowss[ci % 3], gss[ci % 3]
        for j in range(CHUNK_ROWS):
            pltpu.make_async_copy(feat_sh.at[eb.at[j, 0]], rows.at[j], sem).wait()

    def fire_scatters(ci):
        eb, rows, sem = ebs[ci % 3], rowss[ci % 3], sss[ci % 3]
        for j in range(CHUNK_ROWS):
            pltpu.async_copy(rows.at[j], acc_sh.at[eb.at[j, 1]], sem, add=True)

    def wait_scatters(ci):
        eb, rows, sem = ebs[ci % 3], rowss[ci % 3], sss[ci % 3]
        for j in range(CHUNK_ROWS):
            pltpu.make_async_copy(rows.at[j], acc_sh.at[eb.at[j, 1]], sem).wait()

    def scale(ci):
        eb, rows = ebs[ci % 3], rowss[ci % 3]

        @plsc.parallel_loop(0, CHUNK_ROWS * 8, unroll=2)
        def _(g):
            j = g >> 3
            e0 = (g & 7) * 16
            wv = plsc.bitcast(eb[j, 2, pl.ds(e0, 16)], jnp.float32)
            for k in range(16):
                rows[j, e0 + k, :] = rows[j, e0 + k, :] * wv[k]

    idx_copy(0)
    fire_gathers(0)
    for ci in range(NCHUNK):
        wait_gathers(ci)
        if ci >= 2:
            wait_scatters(ci - 2)
        if ci + 1 < NCHUNK:
            idx_copy(ci + 1)
            fire_gathers(ci + 1)
        scale(ci)
        fire_scatters(ci)
    wait_scatters(NCHUNK - 2)
    wait_scatters(NCHUNK - 1)

    plsc.subcore_barrier()
    pltpu.sync_copy(
        acc_sh.at[pl.ds(node0, NODES_PER_TILE)],
        out_hbm.at[c, pl.ds(node0, NODES_PER_TILE)],
    )


PROWS = N_PAD // 8
_BLK = 128
_GRID = PROWS // _BLK


def _tc1_body(x_ref, w_ref, deg_ref, o_ref):
    dis = lax.rsqrt(deg_ref[0] + deg_ref[1] + 1.0)
    xp = jnp.concatenate(
        [x_ref[...], jnp.zeros((N_PAD - N, D_IN), jnp.float32)], axis=0)
    xv = xp.reshape(PROWS, 8 * D_IN)
    h = jnp.dot(xv, w_ref[...], preferred_element_type=jnp.float32)
    o_ref[...] = h * dis


def _tc2_body(p_ref, h_ref, deg_ref, b_ref, o_ref):
    dis = lax.rsqrt(deg_ref[0] + deg_ref[1] + 1.0)
    agg = dis * (p_ref[0] + p_ref[1] + h_ref[...]) + b_ref[...]
    o_ref[...] = dis * jnp.maximum(agg, 0.0)


def _tc3_body(q_ref, r_ref, deg_ref, w_ref, b_ref, o_ref):
    dis = lax.rsqrt(deg_ref[0] + deg_ref[1] + 1.0)
    agg = dis * (q_ref[0] + q_ref[1] + r_ref[...])
    y = jnp.dot(agg, w_ref[...], preferred_element_type=jnp.float32) + b_ref[...]
    y = y.reshape(PROWS, 8, D_OUT)
    m = jnp.max(y, axis=2, keepdims=True)
    lse = m + jnp.log(jnp.sum(jnp.exp(y - m), axis=2, keepdims=True))
    o_ref[...] = (y - lse).reshape(N_PAD, D_OUT)[:N]


def _tc1(x, W1b, deg_parts):
    return pl.pallas_call(
        _tc1_body,
        out_shape=jax.ShapeDtypeStruct((PROWS, 128), jnp.float32),
    )(x, W1b, deg_parts)


def _tc2(parts, h1p, deg_parts, b1p):
    return pl.pallas_call(
        _tc2_body,
        out_shape=jax.ShapeDtypeStruct((PROWS, 128), jnp.float32),
    )(parts, h1p, deg_parts, b1p)


def _tc3(parts, rp, deg_parts, W2b, b2p):
    return pl.pallas_call(
        _tc3_body,
        out_shape=jax.ShapeDtypeStruct((N, D_OUT), jnp.float32),
    )(parts, rp, deg_parts, W2b, b2p)


def kernel(x, edge_index, edge_weight, W1, b1, W2, b2):
    src = edge_index[0]
    dst = edge_index[1]

    npad = E_PAD - E
    pad_idx = N + (jnp.arange(npad, dtype=jnp.int32) % (N_PAD - N))
    src2 = jnp.concatenate([src, pad_idx]).reshape(EROWS, 128)
    dst2 = jnp.concatenate([dst, pad_idx]).reshape(EROWS, 128)
    ew_pad = jnp.concatenate([edge_weight, jnp.zeros((npad,), jnp.float32)])
    ew2 = ew_pad.reshape(EROWS, 128)
    edges = jnp.stack(
        [src2, dst2, jax.lax.bitcast_convert_type(ew2, jnp.int32)], axis=1)

    zeros16 = jnp.zeros((NODES_PER_TILE, D_HID), jnp.float32)
    W1b = jnp.kron(jnp.eye(8, dtype=jnp.float32), W1)
    W2b = jnp.kron(jnp.eye(8, dtype=jnp.float32), W2)
    b1p = jnp.tile(b1, 8).reshape(1, 128)
    b2p = jnp.tile(b2, 8).reshape(1, 8 * D_OUT)

    deg_parts = _sc_deg(edges, zeros16).reshape(2, PROWS, 128)
    h1p = _tc1(x, W1b, deg_parts)
    parts1 = _sc_edge(edges, h1p.reshape(N_PAD, D_HID), zeros16)
    rp = _tc2(parts1.reshape(2, PROWS, 128), h1p, deg_parts, b1p)
    parts2 = _sc_edge(edges, rp.reshape(N_PAD, D_HID), zeros16)
    return _tc3(parts2.reshape(2, PROWS, 128), rp, deg_parts, W2b, b2p)
